# baseline (speedup 1.0000x reference)
import functools

import jax
import jax.numpy as jnp
from jax import lax
from jax.experimental import pallas as pl
from jax.experimental.pallas import tpu as pltpu
from jax.experimental.pallas import tpu_sc as plsc

N = 10000
E = 160000
D_IN = 7
D_H = 640
D_OUT = 80

NC = 2
NS = 16
L = 16
DP = 128

NPAD = 10240
EPT = E // NS
ZB = 16
EB = 2000


def _make_agg(QD, chunks_per_core, C, RB, ZBK, DEPTH, layer1):
    rsub = C * QD // NS
    assert C % NS == 0 and rsub % ZBK == 0 and RB % L == 0 and RB * QD <= 128
    assert ZBK <= RB * QD and 8 * QD <= RB * QD
    PF = 8192
    assert NPAD * PF < 2**31 and (C + 8) < PF
    mesh = plsc.VectorSubcoreMesh(core_axis_name="c", subcore_axis_name="s")
    MB = (EPT + 2 * RB + 2 * L + RB - 1) // RB
    mlen = MB * RB
    dump = mlen - L

    @functools.partial(
        pl.kernel,
        mesh=mesh,
        compiler_params=pltpu.CompilerParams(needs_layout_passes=False),
        out_type=jax.ShapeDtypeStruct((NPAD * QD, 128), jnp.float32),
        scratch_types=[
            pltpu.VMEM((EB,), jnp.int32),
            pltpu.VMEM((EB,), jnp.int32),
            pltpu.VMEM((EB,), jnp.float32),
            pltpu.VMEM((mlen,), jnp.int32),
            pltpu.VMEM((mlen + L,), jnp.float32),
        ] + [pltpu.VMEM((RB * QD,), jnp.int32) for _ in range(2 * DEPTH)]
          + [pltpu.VMEM((RB * QD, 128), jnp.float32) for _ in range(DEPTH)]
          + [pltpu.VMEM_SHARED(((C + 8) * QD, 128), jnp.float32)]
          + [pltpu.SemaphoreType.DMA for _ in range(2 * DEPTH)],
    )
    def agg(table, src, dst, w, out, dst_v, src_v, w_v, m_sd, m_w, *rest):
        idxs = rest[:2 * DEPTH]
        gbufs = rest[2 * DEPTH:3 * DEPTH]
        acc = rest[3 * DEPTH]
        sems = rest[3 * DEPTH + 1:]
        slots = [(idxs[2 * i], idxs[2 * i + 1], gbufs[i], sems[i],
                  sems[DEPTH + i]) for i in range(DEPTH)]
        gbuf_a = gbufs[0]
        cid = lax.axis_index("c")
        sid = lax.axis_index("s")
        ebase = sid * EPT
        iota = lax.iota(jnp.int32, L)
        zvec_f = jnp.zeros((L,), jnp.float32)
        zvec_i = jnp.zeros((L,), jnp.int32)

        def chunk_body(ch, carry):
            lo = (cid * chunks_per_core + ch) * C

            def zg(j, c):
                for kk in range(128 // L):
                    gbuf_a[j, pl.ds(kk * L, L)] = zvec_f
                return c
            lax.fori_loop(0, RB * QD, zg, 0)

            def zacc(t, c):
                pltpu.sync_copy(gbuf_a.at[pl.ds(0, ZBK)],
                                acc.at[pl.ds(sid * rsub + t * ZBK, ZBK)])
                return c
            lax.fori_loop(0, rsub // ZBK, zacc, 0)

            @pl.when(sid == 0)
            def _():
                pltpu.sync_copy(gbuf_a.at[pl.ds(0, 8 * QD)],
                                acc.at[pl.ds(C * QD, 8 * QD)])
            plsc.subcore_barrier()

            def blk_body(blk, m):
                pltpu.sync_copy(dst.at[pl.ds(ebase + blk * EB, EB)], dst_v)
                pltpu.sync_copy(src.at[pl.ds(ebase + blk * EB, EB)], src_v)
                pltpu.sync_copy(w.at[pl.ds(ebase + blk * EB, EB)], w_v)

                def filt(g, m):
                    dvec = dst_v[pl.ds(g * L, L)]
                    mask = (dvec >= lo) & (dvec < lo + C)
                    pos = plsc.cumsum(mask.astype(jnp.int32))
                    tgt = jnp.where(mask, m + pos - 1, dump + iota)
                    packed = (lax.shift_left(src_v[pl.ds(g * L, L)], 13)
                              + (dvec - lo))
                    plsc.store_scatter(m_sd, [tgt], packed)
                    plsc.store_scatter(m_w, [tgt], w_v[pl.ds(g * L, L)])
                    return m + pos[L - 1]
                return lax.fori_loop(0, EB // L, filt, m)
            m = lax.fori_loop(0, EPT // EB, blk_body, jnp.int32(0))

            for t in range(RB // L):
                m_sd[pl.ds(m + t * L, L)] = jnp.full((L,), C, jnp.int32)
                m_w[pl.ds(m + t * L, L)] = zvec_f
            nb = (m + RB - 1) // RB

            def bidx(base, sidx, didx):
                for g in range(RB // L):
                    if QD == 1:
                        v = m_sd[pl.ds(base + g * L, L)]
                        sidx[pl.ds(g * L, L)] = (
                            lax.shift_right_logical(v, 13))
                        didx[pl.ds(g * L, L)] = lax.bitwise_and(v, 8191)
                    else:
                        for t in range(QD):
                            pos5 = t * L + iota
                            lane = base + g * L + pos5 // QD
                            qoff = pos5 % QD
                            v = plsc.load_gather(m_sd, [lane])
                            sg = lax.shift_right_logical(v, 13)
                            dg = lax.bitwise_and(v, 8191)
                            sidx[pl.ds(g * QD * L + t * L, L)] = (
                                sg * QD + qoff)
                            didx[pl.ds(g * QD * L + t * L, L)] = (
                                dg * QD + qoff)

            def scale(base, gbuf):
                for g in range(RB // L):
                    wv = m_w[pl.ds(base + g * L, L)]
                    for j in range(L):
                        ws = wv[j]
                        row = g * L + j
                        if layer1:
                            scale_v = jnp.where(iota < D_IN, ws, 1.0
                                                ).astype(jnp.float32)
                            gbuf[row, pl.ds(0, L)] = (
                                gbuf[row, pl.ds(0, L)] * scale_v)
                        else:
                            for q in range(QD):
                                for kk in range(128 // L):
                                    gbuf[row * QD + q, pl.ds(kk * L, L)] = (
                                        gbuf[row * QD + q,
                                             pl.ds(kk * L, L)] * ws)

            def step(b, cur, nxt):
                sidx_p, didx_p, gbuf_p, sem_p, sem_sp = cur
                sidx_q, didx_q, gbuf_q, sem_q, sem_sq = nxt
                pltpu.make_async_copy(table.at[sidx_p], gbuf_p,
                                      sem_p).wait()
                @pl.when(b + 1 < nb)
                def _():
                    @pl.when(b >= DEPTH - 1)
                    def _():
                        pltpu.make_async_copy(gbuf_q, acc.at[didx_q],
                                              sem_sq).wait()
                    bidx((b + 1) * RB, sidx_q, didx_q)
                    pltpu.async_copy(table.at[sidx_q], gbuf_q, sem_q)
                scale(b * RB, gbuf_p)
                pltpu.async_copy(gbuf_p, acc.at[didx_p], sem_sp, add=True)

            @pl.when(nb > 0)
            def _():
                bidx(0, slots[0][0], slots[0][1])
                pltpu.async_copy(table.at[slots[0][0]], slots[0][2],
                                 slots[0][3])

            def batch(b, c):
                for p in range(DEPTH):
                    @pl.when(b % DEPTH == p)
                    def _():
                        step(b, slots[p], slots[(p + 1) % DEPTH])
                return c
            lax.fori_loop(0, nb, batch, 0)
            for back in range(DEPTH, 0, -1):
                for p in range(DEPTH):
                    @pl.when((nb >= back) & ((nb - back) % DEPTH == p))
                    def _():
                        pltpu.make_async_copy(slots[p][2],
                                              acc.at[slots[p][1]],
                                              slots[p][4]).wait()
            plsc.subcore_barrier()

            def outc(t, c):
                pltpu.sync_copy(
                    acc.at[pl.ds(sid * rsub + t * ZBK, ZBK)],
                    out.at[pl.ds(lo * QD + sid * rsub + t * ZBK, ZBK)])
                return c
            lax.fori_loop(0, rsub // ZBK, outc, 0)
            plsc.subcore_barrier()
            return carry
        lax.fori_loop(0, chunks_per_core, chunk_body, 0)

    return agg


_agg1 = _make_agg(1, 1, NPAD // NC, 128, 64, 2, layer1=True)
_agg2 = _make_agg(5, 4, NPAD // (4 * NC), 16, 80, 2, layer1=False)

BM = 256


def _tc1_body(acc_ref, x_ref, wl_ref, wr_ref, b_ref, o_ref):
    acc = acc_ref[...]
    deg = jnp.clip(acc[:, D_IN:D_IN + 1], 1.0, None)
    aggn = acc / deg
    z = (jnp.dot(aggn, wl_ref[...], preferred_element_type=jnp.float32)
         + jnp.dot(x_ref[...], wr_ref[...], preferred_element_type=jnp.float32)
         + b_ref[...])
    o_ref[...] = jnp.maximum(z, 0.0)


def _tc1(acc1, x_pad, wl, wr, b):
    return pl.pallas_call(
        _tc1_body,
        grid=(NPAD // BM,),
        in_specs=[
            pl.BlockSpec((BM, DP), lambda i: (i, 0)),
            pl.BlockSpec((BM, DP), lambda i: (i, 0)),
            pl.BlockSpec((DP, D_H), lambda i: (0, 0)),
            pl.BlockSpec((DP, D_H), lambda i: (0, 0)),
            pl.BlockSpec((1, D_H), lambda i: (0, 0)),
        ],
        out_specs=pl.BlockSpec((BM, D_H), lambda i: (i, 0)),
        out_shape=jax.ShapeDtypeStruct((NPAD, D_H), jnp.float32),
    )(acc1, x_pad, wl, wr, b)


def _tc2_body(acc2_ref, acc1_ref, h1_ref, w2l_ref, w2r_ref, we_ref, b2_ref,
              be_ref, o_ref):
    deg = jnp.clip(acc1_ref[...][:, D_IN:D_IN + 1], 1.0, None)
    aggn = acc2_ref[...] / deg
    z = (jnp.dot(aggn, w2l_ref[...], preferred_element_type=jnp.float32)
         + jnp.dot(h1_ref[...], w2r_ref[...], preferred_element_type=jnp.float32)
         + b2_ref[...])
    h2 = jnp.maximum(z, 0.0)
    o_ref[...] = (jnp.dot(h2, we_ref[...], preferred_element_type=jnp.float32)
                  + be_ref[...])


def _tc2(acc2, acc1, h1, w2l, w2r, we, b2, be):
    return pl.pallas_call(
        _tc2_body,
        grid=(NPAD // BM,),
        in_specs=[
            pl.BlockSpec((BM, D_H), lambda i: (i, 0)),
            pl.BlockSpec((BM, DP), lambda i: (i, 0)),
            pl.BlockSpec((BM, D_H), lambda i: (i, 0)),
            pl.BlockSpec((D_H, D_H), lambda i: (0, 0)),
            pl.BlockSpec((D_H, D_H), lambda i: (0, 0)),
            pl.BlockSpec((D_H, D_OUT), lambda i: (0, 0)),
            pl.BlockSpec((1, D_H), lambda i: (0, 0)),
            pl.BlockSpec((1, D_OUT), lambda i: (0, 0)),
        ],
        out_specs=pl.BlockSpec((BM, D_OUT), lambda i: (i, 0)),
        out_shape=jax.ShapeDtypeStruct((NPAD, D_OUT), jnp.float32),
    )(acc2, acc1, h1, w2l, w2r, we, b2, be)


def kernel(x, edge_index, edge_weight, W1l, b1, W1r, W2l, b2, W2r, We, be):
    src = edge_index[0]
    dst = edge_index[1]
    x_pad = jnp.pad(
        jnp.concatenate(
            [x, jnp.ones((N, 1), jnp.float32),
             jnp.zeros((N, DP - D_IN - 1), jnp.float32)], axis=1),
        ((0, NPAD - N), (0, 0)))
    acc1 = _agg1(x_pad, src, dst, edge_weight)
    w1l_p = jnp.pad(W1l, ((0, DP - D_IN), (0, 0)))
    w1r_p = jnp.pad(W1r, ((0, DP - D_IN), (0, 0)))
    h1 = _tc1(acc1, x_pad, w1l_p, w1r_p, b1.reshape(1, -1))
    acc2 = _agg2(h1.reshape(NPAD * 5, 128), src, dst,
                 edge_weight).reshape(NPAD, D_H)
    out = _tc2(acc2, acc1, h1, W2l, W2r, We,
               b2.reshape(1, -1), be.reshape(1, -1))
    return out[:N]


# speedup vs baseline: 1.0209x; 1.0209x over previous
import functools

import jax
import jax.numpy as jnp
from jax import lax
from jax.experimental import pallas as pl
from jax.experimental.pallas import tpu as pltpu
from jax.experimental.pallas import tpu_sc as plsc

N = 10000
E = 160000
D_IN = 7
D_H = 640
D_OUT = 80

NC = 2
NS = 16
L = 16
DP = 128

NPAD = 10240
EPT = E // NS
ZB = 16
EB = 2000


def _make_agg(QD, chunks_per_core, C, RB, ZBK, DEPTH, layer1):
    rsub = C * QD // NS
    NH = (RB * QD + 127) // 128
    HS = RB * QD // NH
    assert C % NS == 0 and rsub % ZBK == 0 and RB % L == 0
    assert RB * QD % NH == 0 and HS <= 128 and (QD * L) % HS in (0, QD * L)
    assert HS % (QD * L) == 0
    assert ZBK <= RB * QD and 8 * QD <= RB * QD
    PF = 8192
    assert NPAD * PF < 2**31 and (C + 8) < PF
    mesh = plsc.VectorSubcoreMesh(core_axis_name="c", subcore_axis_name="s")
    MB = (EPT + 2 * RB + 2 * L + RB - 1) // RB
    mlen = MB * RB
    dump = mlen - L

    @functools.partial(
        pl.kernel,
        mesh=mesh,
        compiler_params=pltpu.CompilerParams(needs_layout_passes=False),
        out_type=jax.ShapeDtypeStruct((NPAD * QD, 128), jnp.float32),
        scratch_types=[
            pltpu.VMEM((EB,), jnp.int32),
            pltpu.VMEM((EB,), jnp.int32),
            pltpu.VMEM((EB,), jnp.float32),
            pltpu.VMEM((mlen,), jnp.int32),
            pltpu.VMEM((mlen + L,), jnp.float32),
        ] + [pltpu.VMEM((HS,), jnp.int32) for _ in range(2 * NH * DEPTH)]
          + [pltpu.VMEM((RB * QD, 128), jnp.float32) for _ in range(DEPTH)]
          + [pltpu.VMEM_SHARED(((C + 8) * QD, 128), jnp.float32)]
          + [pltpu.SemaphoreType.DMA for _ in range(2 * DEPTH)],
    )
    def agg(table, src, dst, w, out, dst_v, src_v, w_v, m_sd, m_w, *rest):
        idxs = rest[:2 * NH * DEPTH]
        gbufs = rest[2 * NH * DEPTH:2 * NH * DEPTH + DEPTH]
        acc = rest[2 * NH * DEPTH + DEPTH]
        sems = rest[2 * NH * DEPTH + DEPTH + 1:]
        slots = [(list(idxs[2 * NH * i:2 * NH * i + NH]),
                  list(idxs[2 * NH * i + NH:2 * NH * i + 2 * NH]),
                  gbufs[i], sems[i], sems[DEPTH + i]) for i in range(DEPTH)]
        gbuf_a = gbufs[0]
        cid = lax.axis_index("c")
        sid = lax.axis_index("s")
        ebase = sid * EPT
        iota = lax.iota(jnp.int32, L)
        zvec_f = jnp.zeros((L,), jnp.float32)
        zvec_i = jnp.zeros((L,), jnp.int32)

        def chunk_body(ch, carry):
            lo = (cid * chunks_per_core + ch) * C

            def zg(j, c):
                for kk in range(128 // L):
                    gbuf_a[j, pl.ds(kk * L, L)] = zvec_f
                return c
            lax.fori_loop(0, RB * QD, zg, 0)

            def zacc(t, c):
                pltpu.sync_copy(gbuf_a.at[pl.ds(0, ZBK)],
                                acc.at[pl.ds(sid * rsub + t * ZBK, ZBK)])
                return c
            lax.fori_loop(0, rsub // ZBK, zacc, 0)

            @pl.when(sid == 0)
            def _():
                pltpu.sync_copy(gbuf_a.at[pl.ds(0, 8 * QD)],
                                acc.at[pl.ds(C * QD, 8 * QD)])
            plsc.subcore_barrier()

            def blk_body(blk, m):
                pltpu.sync_copy(dst.at[pl.ds(ebase + blk * EB, EB)], dst_v)
                pltpu.sync_copy(src.at[pl.ds(ebase + blk * EB, EB)], src_v)
                pltpu.sync_copy(w.at[pl.ds(ebase + blk * EB, EB)], w_v)

                def filt(g, m):
                    dvec = dst_v[pl.ds(g * L, L)]
                    mask = (dvec >= lo) & (dvec < lo + C)
                    pos = plsc.cumsum(mask.astype(jnp.int32))
                    tgt = jnp.where(mask, m + pos - 1, dump + iota)
                    packed = (lax.shift_left(src_v[pl.ds(g * L, L)], 13)
                              + (dvec - lo))
                    plsc.store_scatter(m_sd, [tgt], packed)
                    plsc.store_scatter(m_w, [tgt], w_v[pl.ds(g * L, L)])
                    return m + pos[L - 1]
                return lax.fori_loop(0, EB // L, filt, m)
            m = lax.fori_loop(0, EPT // EB, blk_body, jnp.int32(0))

            for t in range(RB // L):
                m_sd[pl.ds(m + t * L, L)] = jnp.full((L,), C, jnp.int32)
                m_w[pl.ds(m + t * L, L)] = zvec_f
            nb = (m + RB - 1) // RB

            def bidx(base, sidxs, didxs):
                for g in range(RB // L):
                    h = (g * QD * L) // HS
                    off = (g * QD * L) % HS
                    if QD == 1:
                        v = m_sd[pl.ds(base + g * L, L)]
                        sidxs[h][pl.ds(off, L)] = (
                            lax.shift_right_logical(v, 13))
                        didxs[h][pl.ds(off, L)] = lax.bitwise_and(v, 8191)
                    else:
                        for t in range(QD):
                            pos5 = t * L + iota
                            lane = base + g * L + pos5 // QD
                            qoff = pos5 % QD
                            v = plsc.load_gather(m_sd, [lane])
                            sg = lax.shift_right_logical(v, 13)
                            dg = lax.bitwise_and(v, 8191)
                            sidxs[h][pl.ds(off + t * L, L)] = (
                                sg * QD + qoff)
                            didxs[h][pl.ds(off + t * L, L)] = (
                                dg * QD + qoff)

            def scale(base, gbuf):
                def srow(j, cc):
                    wv = m_w[pl.ds(base + j, L)]
                    ws = wv[0]
                    if layer1:
                        scale_v = jnp.where(iota < D_IN, ws, 1.0
                                            ).astype(jnp.float32)
                        gbuf[j, pl.ds(0, L)] = (
                            gbuf[j, pl.ds(0, L)] * scale_v)
                    else:
                        for q in range(QD):
                            for kk in range(128 // L):
                                gbuf[j * QD + q, pl.ds(kk * L, L)] = (
                                    gbuf[j * QD + q, pl.ds(kk * L, L)] * ws)
                    return cc
                lax.fori_loop(0, RB, srow, 0)

            def step(b, cur, nxt):
                sidxs_p, didxs_p, gbuf_p, sem_p, sem_sp = cur
                sidxs_q, didxs_q, gbuf_q, sem_q, sem_sq = nxt
                for h in range(NH):
                    pltpu.make_async_copy(
                        table.at[sidxs_p[h]],
                        gbuf_p.at[pl.ds(h * HS, HS)], sem_p).wait()
                @pl.when(b + 1 < nb)
                def _():
                    @pl.when(b >= DEPTH - 1)
                    def _():
                        for h in range(NH):
                            pltpu.make_async_copy(
                                gbuf_q.at[pl.ds(h * HS, HS)],
                                acc.at[didxs_q[h]], sem_sq).wait()
                    bidx((b + 1) * RB, sidxs_q, didxs_q)
                    for h in range(NH):
                        pltpu.async_copy(table.at[sidxs_q[h]],
                                         gbuf_q.at[pl.ds(h * HS, HS)],
                                         sem_q)
                scale(b * RB, gbuf_p)
                for h in range(NH):
                    pltpu.async_copy(gbuf_p.at[pl.ds(h * HS, HS)],
                                     acc.at[didxs_p[h]], sem_sp, add=True)

            @pl.when(nb > 0)
            def _():
                bidx(0, slots[0][0], slots[0][1])
                for h in range(NH):
                    pltpu.async_copy(table.at[slots[0][0][h]],
                                     slots[0][2].at[pl.ds(h * HS, HS)],
                                     slots[0][3])

            def batch(b, c):
                for p in range(DEPTH):
                    @pl.when(b % DEPTH == p)
                    def _():
                        step(b, slots[p], slots[(p + 1) % DEPTH])
                return c
            lax.fori_loop(0, nb, batch, 0)
            for back in range(DEPTH, 0, -1):
                for p in range(DEPTH):
                    @pl.when((nb >= back) & ((nb - back) % DEPTH == p))
                    def _():
                        for h in range(NH):
                            pltpu.make_async_copy(
                                slots[p][2].at[pl.ds(h * HS, HS)],
                                acc.at[slots[p][1][h]],
                                slots[p][4]).wait()
            plsc.subcore_barrier()

            def outc(t, c):
                pltpu.sync_copy(
                    acc.at[pl.ds(sid * rsub + t * ZBK, ZBK)],
                    out.at[pl.ds(lo * QD + sid * rsub + t * ZBK, ZBK)])
                return c
            lax.fori_loop(0, rsub // ZBK, outc, 0)
            plsc.subcore_barrier()
            return carry
        lax.fori_loop(0, chunks_per_core, chunk_body, 0)

    return agg


_agg1 = _make_agg(1, 1, NPAD // NC, 128, 64, 2, layer1=True)
_agg2 = _make_agg(5, 4, NPAD // (4 * NC), 32, 80, 2, layer1=False)

BM = 256


def _tc1_body(acc_ref, x_ref, wl_ref, wr_ref, b_ref, o_ref):
    acc = acc_ref[...]
    deg = jnp.clip(acc[:, D_IN:D_IN + 1], 1.0, None)
    aggn = acc / deg
    z = (jnp.dot(aggn, wl_ref[...], preferred_element_type=jnp.float32)
         + jnp.dot(x_ref[...], wr_ref[...], preferred_element_type=jnp.float32)
         + b_ref[...])
    o_ref[...] = jnp.maximum(z, 0.0)


def _tc1(acc1, x_pad, wl, wr, b):
    return pl.pallas_call(
        _tc1_body,
        grid=(NPAD // BM,),
        in_specs=[
            pl.BlockSpec((BM, DP), lambda i: (i, 0)),
            pl.BlockSpec((BM, DP), lambda i: (i, 0)),
            pl.BlockSpec((DP, D_H), lambda i: (0, 0)),
            pl.BlockSpec((DP, D_H), lambda i: (0, 0)),
            pl.BlockSpec((1, D_H), lambda i: (0, 0)),
        ],
        out_specs=pl.BlockSpec((BM, D_H), lambda i: (i, 0)),
        out_shape=jax.ShapeDtypeStruct((NPAD, D_H), jnp.float32),
    )(acc1, x_pad, wl, wr, b)


def _tc2_body(acc2_ref, acc1_ref, h1_ref, w2l_ref, w2r_ref, we_ref, b2_ref,
              be_ref, o_ref):
    deg = jnp.clip(acc1_ref[...][:, D_IN:D_IN + 1], 1.0, None)
    aggn = acc2_ref[...] / deg
    z = (jnp.dot(aggn, w2l_ref[...], preferred_element_type=jnp.float32)
         + jnp.dot(h1_ref[...], w2r_ref[...], preferred_element_type=jnp.float32)
         + b2_ref[...])
    h2 = jnp.maximum(z, 0.0)
    o_ref[...] = (jnp.dot(h2, we_ref[...], preferred_element_type=jnp.float32)
                  + be_ref[...])


def _tc2(acc2, acc1, h1, w2l, w2r, we, b2, be):
    return pl.pallas_call(
        _tc2_body,
        grid=(NPAD // BM,),
        in_specs=[
            pl.BlockSpec((BM, D_H), lambda i: (i, 0)),
            pl.BlockSpec((BM, DP), lambda i: (i, 0)),
            pl.BlockSpec((BM, D_H), lambda i: (i, 0)),
            pl.BlockSpec((D_H, D_H), lambda i: (0, 0)),
            pl.BlockSpec((D_H, D_H), lambda i: (0, 0)),
            pl.BlockSpec((D_H, D_OUT), lambda i: (0, 0)),
            pl.BlockSpec((1, D_H), lambda i: (0, 0)),
            pl.BlockSpec((1, D_OUT), lambda i: (0, 0)),
        ],
        out_specs=pl.BlockSpec((BM, D_OUT), lambda i: (i, 0)),
        out_shape=jax.ShapeDtypeStruct((NPAD, D_OUT), jnp.float32),
    )(acc2, acc1, h1, w2l, w2r, we, b2, be)


def kernel(x, edge_index, edge_weight, W1l, b1, W1r, W2l, b2, W2r, We, be):
    src = edge_index[0]
    dst = edge_index[1]
    x_pad = jnp.pad(
        jnp.concatenate(
            [x, jnp.ones((N, 1), jnp.float32),
             jnp.zeros((N, DP - D_IN - 1), jnp.float32)], axis=1),
        ((0, NPAD - N), (0, 0)))
    acc1 = _agg1(x_pad, src, dst, edge_weight)
    w1l_p = jnp.pad(W1l, ((0, DP - D_IN), (0, 0)))
    w1r_p = jnp.pad(W1r, ((0, DP - D_IN), (0, 0)))
    h1 = _tc1(acc1, x_pad, w1l_p, w1r_p, b1.reshape(1, -1))
    acc2 = _agg2(h1.reshape(NPAD * 5, 128), src, dst,
                 edge_weight).reshape(NPAD, D_H)
    out = _tc2(acc2, acc1, h1, W2l, W2r, We,
               b2.reshape(1, -1), be.reshape(1, -1))
    return out[:N]
